# baseline (speedup 1.0000x reference)
import functools

import jax
import jax.numpy as jnp
from jax.experimental import pallas as pl
from jax.experimental.pallas import tpu as pltpu


_K = 64
_MAX_ITERS = 34


def _matmul_body(br1, x_ref, w_ref, b_ref, h_ref, rmax_ref, npos_ref):
    i = pl.program_id(1)
    h = jax.lax.dot_general(
        x_ref[pl.ds(i * br1, br1), :], w_ref[...],
        (((1,), (1,)), ((), ())),
        preferred_element_type=jnp.float32,
        precision=jax.lax.Precision.DEFAULT,
    )
    h = h + b_ref[...]
    h_ref[...] = h
    rmax_ref[0, :, :] = jnp.max(h, axis=1, keepdims=True)
    npos_ref[0, :, :] = jnp.sum((h > 0).astype(jnp.float32), axis=1,
                                keepdims=True)


def _select_body(n_chunks, rmax_ref, npos_ref, h_ref, o_ref):
    br, d_sae = o_ref.shape
    cw = d_sae // n_chunks
    tiny = jnp.float32(1e-30)
    lo0 = jnp.full((br, 1), tiny, jnp.float32)
    hi0 = jnp.maximum(jnp.max(rmax_ref[...], axis=0), lo0)
    npos = jnp.sum(npos_ref[...], axis=0)
    res0 = (npos <= _K).astype(jnp.float32)
    kf = jnp.float32(_K)

    def cond(carry):
        i, _, _, res = carry
        return jnp.logical_and(i < _MAX_ITERS, jnp.min(res) < 0.5)

    def body(carry):
        i, lo, hi, res = carry
        mid = 0.5 * (lo + hi)
        accs = [jnp.zeros((br, 128), jnp.float32) for _ in range(8)]
        for c in range(d_sae // 128):
            hc = h_ref[:, c * 128:(c + 1) * 128]
            accs[c % 8] = accs[c % 8] + (hc >= mid).astype(jnp.float32)
        acc = (accs[0] + accs[1]) + (accs[2] + accs[3]) + (
            (accs[4] + accs[5]) + (accs[6] + accs[7]))
        cnt = jnp.sum(acc, axis=1, keepdims=True)
        ge = cnt >= kf
        lo = jnp.where(ge, mid, lo)
        hi = jnp.where(ge, hi, mid)
        res = jnp.where(ge, (cnt == kf).astype(jnp.float32), res)
        return i + 1, lo, hi, res

    _, lo, _, _ = jax.lax.while_loop(
        cond, body, (jnp.int32(0), lo0, hi0, res0))

    for c in range(n_chunks):
        sl = slice(c * cw, (c + 1) * cw)
        hc = h_ref[:, sl]
        o_ref[:, sl] = jnp.where(hc >= lo, hc, 0.0)


def kernel(x, token_mask, W, b):
    batch, seq, d_model = x.shape
    d_sae = W.shape[0]
    n = batch * seq
    xf = x.reshape(n, d_model)

    br1 = min(512, n)
    sae_tile = min(2048, d_sae)
    n_sae_tiles = d_sae // sae_tile
    h, rmax_p, npos_p = pl.pallas_call(
        functools.partial(_matmul_body, br1),
        grid=(n_sae_tiles, n // br1),
        in_specs=[
            pl.BlockSpec((n, d_model), lambda j, i: (0, 0)),
            pl.BlockSpec((sae_tile, d_model), lambda j, i: (j, 0)),
            pl.BlockSpec((1, sae_tile), lambda j, i: (0, j)),
        ],
        out_specs=[
            pl.BlockSpec((br1, sae_tile), lambda j, i: (i, j)),
            pl.BlockSpec((1, br1, 1), lambda j, i: (j, i, 0)),
            pl.BlockSpec((1, br1, 1), lambda j, i: (j, i, 0)),
        ],
        out_shape=[
            jax.ShapeDtypeStruct((n, d_sae), jnp.float32),
            jax.ShapeDtypeStruct((n_sae_tiles, n, 1), jnp.float32),
            jax.ShapeDtypeStruct((n_sae_tiles, n, 1), jnp.float32),
        ],
    )(xf, W, b.reshape(1, d_sae))

    br2 = min(64, n)
    n_chunks = n_sae_tiles
    out = pl.pallas_call(
        functools.partial(_select_body, n_chunks),
        grid=(n // br2,),
        in_specs=[
            pl.BlockSpec((n_sae_tiles, br2, 1), lambda i: (0, i, 0)),
            pl.BlockSpec((n_sae_tiles, br2, 1), lambda i: (0, i, 0)),
            pl.BlockSpec((br2, d_sae), lambda i: (i, 0)),
        ],
        out_specs=pl.BlockSpec((br2, d_sae), lambda i: (i, 0)),
        out_shape=jax.ShapeDtypeStruct((n, d_sae), jnp.float32),
        input_output_aliases={2: 0},
    )(rmax_p, npos_p, h)
    return out.reshape(batch, seq, d_sae)


# speedup vs baseline: 1.1386x; 1.1386x over previous
import functools

import jax
import jax.numpy as jnp
from jax.experimental import pallas as pl
from jax.experimental.pallas import tpu as pltpu


_K = 64
_MAX_ITERS = 34


def _matmul_body(br1, x_ref, w_ref, b_ref, h_ref, rmax_ref, npos_ref):
    i = pl.program_id(1)
    h = jax.lax.dot_general(
        x_ref[pl.ds(i * br1, br1), :], w_ref[...],
        (((1,), (1,)), ((), ())),
        preferred_element_type=jnp.float32,
        precision=jax.lax.Precision.DEFAULT,
    )
    h = h + b_ref[...]
    h_ref[...] = h
    rmax_ref[0, :, :] = jnp.max(h, axis=1, keepdims=True)
    npos_ref[0, :, :] = jnp.sum((h > 0).astype(jnp.float32), axis=1,
                                keepdims=True)


def _select_body(n_chunks, rmax_ref, npos_ref, h_ref, o_ref):
    br, d_sae = o_ref.shape
    cw = d_sae // n_chunks
    tiny = jnp.float32(1e-30)
    lo0 = jnp.full((br, 1), tiny, jnp.float32)
    hi0 = jnp.maximum(jnp.max(rmax_ref[...], axis=0), lo0)
    npos = jnp.sum(npos_ref[...], axis=0)
    res0 = (npos <= _K).astype(jnp.float32)
    kf = jnp.float32(_K)

    def cond(carry):
        i, _, _, res = carry
        return jnp.logical_and(i < _MAX_ITERS, jnp.min(res) < 0.5)

    def body(carry):
        i, lo, hi, res = carry
        mid = 0.5 * (lo + hi)
        accs = [jnp.zeros((br, 128), jnp.float32) for _ in range(8)]
        for c in range(d_sae // 128):
            hc = h_ref[:, c * 128:(c + 1) * 128]
            accs[c % 8] = accs[c % 8] + (hc >= mid).astype(jnp.float32)
        acc = (accs[0] + accs[1]) + (accs[2] + accs[3]) + (
            (accs[4] + accs[5]) + (accs[6] + accs[7]))
        cnt = jnp.sum(acc, axis=1, keepdims=True)
        ge = cnt >= kf
        lo = jnp.where(ge, mid, lo)
        hi = jnp.where(ge, hi, mid)
        res = jnp.where(ge, (cnt == kf).astype(jnp.float32), res)
        return i + 1, lo, hi, res

    _, lo, _, _ = jax.lax.while_loop(
        cond, body, (jnp.int32(0), lo0, hi0, res0))

    for c in range(n_chunks):
        sl = slice(c * cw, (c + 1) * cw)
        hc = h_ref[:, sl]
        o_ref[:, sl] = jnp.where(hc >= lo, hc, 0.0)


def kernel(x, token_mask, W, b):
    batch, seq, d_model = x.shape
    d_sae = W.shape[0]
    n = batch * seq
    xf = x.reshape(n, d_model)

    br1 = min(512, n)
    sae_tile = min(2048, d_sae)
    n_sae_tiles = d_sae // sae_tile
    h, rmax_p, npos_p = pl.pallas_call(
        functools.partial(_matmul_body, br1),
        grid=(n_sae_tiles, n // br1),
        in_specs=[
            pl.BlockSpec((n, d_model), lambda j, i: (0, 0)),
            pl.BlockSpec((sae_tile, d_model), lambda j, i: (j, 0)),
            pl.BlockSpec((1, sae_tile), lambda j, i: (0, j)),
        ],
        out_specs=[
            pl.BlockSpec((br1, sae_tile), lambda j, i: (i, j)),
            pl.BlockSpec((1, br1, 1), lambda j, i: (j, i, 0)),
            pl.BlockSpec((1, br1, 1), lambda j, i: (j, i, 0)),
        ],
        out_shape=[
            jax.ShapeDtypeStruct((n, d_sae), jnp.float32),
            jax.ShapeDtypeStruct((n_sae_tiles, n, 1), jnp.float32),
            jax.ShapeDtypeStruct((n_sae_tiles, n, 1), jnp.float32),
        ],
    )(xf, W, b.reshape(1, d_sae))

    br2 = min(128, n)
    n_chunks = n_sae_tiles
    out = pl.pallas_call(
        functools.partial(_select_body, n_chunks),
        grid=(n // br2,),
        in_specs=[
            pl.BlockSpec((n_sae_tiles, br2, 1), lambda i: (0, i, 0)),
            pl.BlockSpec((n_sae_tiles, br2, 1), lambda i: (0, i, 0)),
            pl.BlockSpec((br2, d_sae), lambda i: (i, 0)),
        ],
        out_specs=pl.BlockSpec((br2, d_sae), lambda i: (i, 0)),
        out_shape=jax.ShapeDtypeStruct((n, d_sae), jnp.float32),
        input_output_aliases={2: 0},
    )(rmax_p, npos_p, h)
    return out.reshape(batch, seq, d_sae)
